# baseline (speedup 1.0000x reference)
import functools

import jax
import jax.numpy as jnp
from jax import lax
from jax.experimental import pallas as pl
from jax.experimental.pallas import tpu as pltpu
from jax.experimental.pallas import tpu_sc as plsc

N = 10000
E = 320000
IN_DIM = 128
H_DIM = 256
OUT_DIM = 256

NC = 2
NS = 16
NW = NC * NS
CHUNK = 128

KPT = 80
EPW = KPT * CHUNK
E_PAD = EPW * NW
GB = 2

N_PAD = 10112
RPT = N_PAD // NS
TRASH = N

R_TC = 400

_MESH = dict(core_axis_name="c", subcore_axis_name="s",
             num_cores=NC, num_subcores=NS)


def _sc_segsum():
  mesh = plsc.VectorSubcoreMesh(**_MESH)
  scratch = (
      [pltpu.VMEM((KPT, CHUNK), jnp.int32)] +
      [pltpu.VMEM((KPT,), jnp.int32)] +
      [pltpu.VMEM((KPT // 2, CHUNK), jnp.int32)] +
      [pltpu.VMEM((CHUNK, 128), jnp.float32)] * GB +
      [pltpu.VMEM_SHARED((N_PAD, 128), jnp.float32)] +
      [pltpu.SemaphoreType.DMA] * 2)

  @functools.partial(
      pl.kernel, mesh=mesh,
      out_type=jax.ShapeDtypeStruct((NC, N_PAD, 128), jnp.float32),
      scratch_types=scratch)
  def k(table, srcs, dsts, out, src_v, ridx_v, dsth_v, *rest):
    rows = rest[:GB]
    acc = rest[GB]
    gsem, ssem = rest[GB + 1], rest[GB + 2]
    c = lax.axis_index("c")
    s = lax.axis_index("s")
    wid = s * NC + c

    iota = lax.iota(jnp.int32, 16)
    wbase = wid * KPT

    def fill(t, carry):
      ridx_v[pl.ds(t * 16, 16)] = iota + (wbase + t * 16)
      return carry

    lax.fori_loop(0, KPT // 16, fill, 0)
    pltpu.async_copy(srcs.at[ridx_v], src_v, gsem).wait()

    z16 = jnp.zeros((16,), jnp.float32)

    def zrow(i, carry):
      for l in range(128 // 16):
        rows[0][i, pl.ds(l * 16, 16)] = z16
      return carry

    lax.fori_loop(0, CHUNK, zrow, 0)
    base = s * RPT
    for t in range(RPT // CHUNK):
      pltpu.sync_copy(rows[0], acc.at[pl.ds(base + t * CHUNK, CHUNK)])
    rem = RPT % CHUNK
    if rem:
      pltpu.sync_copy(rows[0].at[pl.ds(0, rem)],
                      acc.at[pl.ds(base + (RPT // CHUNK) * CHUNK, rem)])
    plsc.subcore_barrier()

    half = KPT // 2
    for h in range(2):
      pltpu.async_copy(dsts.at[ridx_v.at[pl.ds(h * half, half)]], dsth_v,
                       gsem).wait()

      def group(g, carry):
        j0 = h * half + g * GB
        l0 = g * GB
        gds = [pltpu.async_copy(table.at[src_v.at[j0 + b]], rows[b], gsem)
               for b in range(GB)]
        sds = []
        for b in range(GB):
          gds[b].wait()
          sds.append(pltpu.async_copy(rows[b], acc.at[dsth_v.at[l0 + b]],
                                      ssem, add=True))
        for d in sds:
          d.wait()
        return carry

      lax.fori_loop(0, half // GB, group, 0)
    plsc.subcore_barrier()
    pltpu.sync_copy(acc.at[pl.ds(s * RPT, RPT)],
                    out.at[c, pl.ds(s * RPT, RPT)])

  return k


def _tc_layer1(x, sum1, degp, Ws, Wn, b):
  def body(x_ref, s_ref, d_ref, ws_ref, wn_ref, b_ref, o_ref):
    sb = s_ref[0] + s_ref[1]
    deg = d_ref[0, :, :1] + d_ref[1, :, :1]
    recip = 1.0 / jnp.maximum(deg, 1.0)
    h = jnp.dot(x_ref[...], ws_ref[...], preferred_element_type=jnp.float32,
                   precision=lax.Precision.HIGHEST)
    h = h + jnp.dot(sb, wn_ref[...], preferred_element_type=jnp.float32,
                   precision=lax.Precision.HIGHEST) * recip
    h = h + b_ref[...]
    h = jnp.maximum(h, 0.0)
    o_ref[0] = h[:, :128]
    o_ref[1] = h[:, 128:]

  grid = (N // R_TC,)
  return pl.pallas_call(
      body,
      grid=grid,
      in_specs=[
          pl.BlockSpec((R_TC, IN_DIM), lambda i: (i, 0)),
          pl.BlockSpec((NC, R_TC, 128), lambda i: (0, i, 0)),
          pl.BlockSpec((NC, R_TC, 128), lambda i: (0, i, 0)),
          pl.BlockSpec((IN_DIM, H_DIM), lambda i: (0, 0)),
          pl.BlockSpec((IN_DIM, H_DIM), lambda i: (0, 0)),
          pl.BlockSpec((1, H_DIM), lambda i: (0, 0)),
      ],
      out_specs=pl.BlockSpec((NC, R_TC, 128), lambda i: (0, i, 0)),
      out_shape=jax.ShapeDtypeStruct((NC, N, 128), jnp.float32),
  )(x, sum1, degp, Ws, Wn, b)


def _tc_layer2(hst, s2a, s2b, degp, Ws2a, Ws2b, Wn2a, Wn2b, b):
  def body(h_ref, sa_ref, sb_ref, d_ref, wsa_ref, wsb_ref, wna_ref, wnb_ref,
           b_ref, o_ref):
    deg = d_ref[0, :, :1] + d_ref[1, :, :1]
    recip = 1.0 / jnp.maximum(deg, 1.0)
    acc = jnp.dot(h_ref[0], wsa_ref[...], preferred_element_type=jnp.float32,
                   precision=lax.Precision.HIGHEST)
    acc = acc + jnp.dot(h_ref[1], wsb_ref[...],
                        preferred_element_type=jnp.float32,
                   precision=lax.Precision.HIGHEST)
    sa = sa_ref[0] + sa_ref[1]
    sb = sb_ref[0] + sb_ref[1]
    nei = jnp.dot(sa, wna_ref[...], preferred_element_type=jnp.float32,
                   precision=lax.Precision.HIGHEST)
    nei = nei + jnp.dot(sb, wnb_ref[...], preferred_element_type=jnp.float32,
                   precision=lax.Precision.HIGHEST)
    o_ref[...] = acc + nei * recip + b_ref[...]

  grid = (N // R_TC,)
  return pl.pallas_call(
      body,
      grid=grid,
      in_specs=[
          pl.BlockSpec((NC, R_TC, 128), lambda i: (0, i, 0)),
          pl.BlockSpec((NC, R_TC, 128), lambda i: (0, i, 0)),
          pl.BlockSpec((NC, R_TC, 128), lambda i: (0, i, 0)),
          pl.BlockSpec((NC, R_TC, 128), lambda i: (0, i, 0)),
          pl.BlockSpec((128, OUT_DIM), lambda i: (0, 0)),
          pl.BlockSpec((128, OUT_DIM), lambda i: (0, 0)),
          pl.BlockSpec((128, OUT_DIM), lambda i: (0, 0)),
          pl.BlockSpec((128, OUT_DIM), lambda i: (0, 0)),
          pl.BlockSpec((1, OUT_DIM), lambda i: (0, 0)),
      ],
      out_specs=pl.BlockSpec((R_TC, OUT_DIM), lambda i: (i, 0)),
      out_shape=jax.ShapeDtypeStruct((N, OUT_DIM), jnp.float32),
  )(hst, s2a, s2b, degp, Ws2a, Ws2b, Wn2a, Wn2b, b)


def kernel(feats, edge_index, W_self1, W_neigh1, b1, W_self2, W_neigh2, b2):
  src = edge_index[0]
  dst = edge_index[1]
  pad = E_PAD - E
  src_p = jnp.concatenate([src, jnp.zeros((pad,), jnp.int32)])
  dst_p = jnp.concatenate([dst, jnp.full((pad,), TRASH, jnp.int32)])

  srcs = src_p.reshape(NW * KPT, CHUNK)
  dsts = dst_p.reshape(NW * KPT, CHUNK)
  srcs_hi = srcs + N

  ones_tab = jnp.ones((CHUNK, 128), jnp.float32)

  degp = _sc_segsum()(ones_tab, jnp.zeros_like(srcs), dsts)
  sum1 = _sc_segsum()(feats, srcs, dsts)

  hst = _tc_layer1(feats, sum1, degp, W_self1, W_neigh1, b1.reshape(1, H_DIM))

  table2 = hst.reshape(2 * N, 128)
  s2a = _sc_segsum()(table2, srcs, dsts)
  s2b = _sc_segsum()(table2, srcs_hi, dsts)

  out = _tc_layer2(hst, s2a, s2b, degp,
                   W_self2[:128], W_self2[128:],
                   W_neigh2[:128], W_neigh2[128:],
                   b2.reshape(1, OUT_DIM))
  return out


# speedup vs baseline: 8.1993x; 8.1993x over previous
import functools

import jax
import jax.numpy as jnp
from jax import lax
from jax.experimental import pallas as pl
from jax.experimental.pallas import tpu as pltpu
from jax.experimental.pallas import tpu_sc as plsc

N = 10000
E = 320000
IN_DIM = 128
H_DIM = 256
OUT_DIM = 256

NC = 2
NS = 16
NW = NC * NS
CHUNK = 128

KPT = 80
EPW = KPT * CHUNK
E_PAD = EPW * NW
GB = 2

N_PAD = 10112
RPT = N_PAD // NS
TRASH = N

R_TC = 400

_MESH = dict(core_axis_name="c", subcore_axis_name="s",
             num_cores=NC, num_subcores=NS)


def _sc_segsum(with_gather=True):
  mesh = plsc.VectorSubcoreMesh(**_MESH)
  scratch = (
      [pltpu.VMEM((KPT, CHUNK), jnp.int32)] +
      [pltpu.VMEM((KPT,), jnp.int32)] +
      [pltpu.VMEM((KPT // 2, CHUNK), jnp.int32)] +
      [pltpu.VMEM((CHUNK, 128), jnp.float32)] * GB +
      [pltpu.VMEM_SHARED((N_PAD, 128), jnp.float32)] +
      [pltpu.SemaphoreType.DMA] * 2)

  @functools.partial(
      pl.kernel, mesh=mesh,
      out_type=jax.ShapeDtypeStruct((NC, N_PAD, 128), jnp.float32),
      scratch_types=scratch)
  def k(*refs):
    if with_gather:
      (table, srcs, dsts, out, src_v, ridx_v, dsth_v, *rest) = refs
    else:
      (dsts, out, src_v, ridx_v, dsth_v, *rest) = refs
    rows = rest[:GB]
    acc = rest[GB]
    gsem, ssem = rest[GB + 1], rest[GB + 2]
    c = lax.axis_index("c")
    s = lax.axis_index("s")
    wid = s * NC + c

    iota = lax.iota(jnp.int32, 16)
    wbase = wid * KPT

    def fill(t, carry):
      ridx_v[pl.ds(t * 16, 16)] = iota + (wbase + t * 16)
      return carry

    lax.fori_loop(0, KPT // 16, fill, 0)
    if with_gather:
      pltpu.async_copy(srcs.at[ridx_v], src_v, gsem).wait()

    z16 = jnp.zeros((16,), jnp.float32)

    def zrow(i, carry):
      for l in range(128 // 16):
        rows[0][i, pl.ds(l * 16, 16)] = z16
      return carry

    lax.fori_loop(0, CHUNK, zrow, 0)
    base = s * RPT
    for t in range(RPT // CHUNK):
      pltpu.sync_copy(rows[0], acc.at[pl.ds(base + t * CHUNK, CHUNK)])
    rem = RPT % CHUNK
    if rem:
      pltpu.sync_copy(rows[0].at[pl.ds(0, rem)],
                      acc.at[pl.ds(base + (RPT // CHUNK) * CHUNK, rem)])
    if not with_gather:
      one16 = jnp.ones((16,), jnp.float32)

      def orow(i, carry):
        for l in range(128 // 16):
          rows[0][i, pl.ds(l * 16, 16)] = one16
        return carry

      lax.fori_loop(0, CHUNK, orow, 0)
    plsc.subcore_barrier()

    half = KPT // 2
    for h in range(2):
      pltpu.async_copy(dsts.at[ridx_v.at[pl.ds(h * half, half)]], dsth_v,
                       gsem).wait()

      def group(g, carry):
        j0 = h * half + g * GB
        l0 = g * GB
        if with_gather:
          gds = [pltpu.async_copy(table.at[src_v.at[j0 + b]], rows[b], gsem)
                 for b in range(GB)]
        sds = []
        for b in range(GB):
          if with_gather:
            gds[b].wait()
            sds.append(pltpu.async_copy(rows[b], acc.at[dsth_v.at[l0 + b]],
                                        ssem, add=True))
          else:
            sds.append(pltpu.async_copy(rows[0], acc.at[dsth_v.at[l0 + b]],
                                        ssem, add=True))
        for d in sds:
          d.wait()
        return carry

      lax.fori_loop(0, half // GB, group, 0)
    plsc.subcore_barrier()
    pltpu.sync_copy(acc.at[pl.ds(s * RPT, RPT)],
                    out.at[c, pl.ds(s * RPT, RPT)])

  return k


def _tc_layer1(x, sum1, degp, Ws, Wn, b):
  def body(x_ref, s_ref, d_ref, ws_ref, wn_ref, b_ref, o_ref):
    sb = s_ref[0] + s_ref[1]
    deg = d_ref[0, :, :1] + d_ref[1, :, :1]
    recip = 1.0 / jnp.maximum(deg, 1.0)
    h = jnp.dot(x_ref[...], ws_ref[...], preferred_element_type=jnp.float32,
                   precision=lax.Precision.HIGHEST)
    h = h + jnp.dot(sb, wn_ref[...], preferred_element_type=jnp.float32,
                   precision=lax.Precision.HIGHEST) * recip
    h = h + b_ref[...]
    h = jnp.maximum(h, 0.0)
    o_ref[0] = h[:, :128]
    o_ref[1] = h[:, 128:]

  grid = (N // R_TC,)
  return pl.pallas_call(
      body,
      grid=grid,
      in_specs=[
          pl.BlockSpec((R_TC, IN_DIM), lambda i: (i, 0)),
          pl.BlockSpec((NC, R_TC, 128), lambda i: (0, i, 0)),
          pl.BlockSpec((NC, R_TC, 128), lambda i: (0, i, 0)),
          pl.BlockSpec((IN_DIM, H_DIM), lambda i: (0, 0)),
          pl.BlockSpec((IN_DIM, H_DIM), lambda i: (0, 0)),
          pl.BlockSpec((1, H_DIM), lambda i: (0, 0)),
      ],
      out_specs=pl.BlockSpec((NC, R_TC, 128), lambda i: (0, i, 0)),
      out_shape=jax.ShapeDtypeStruct((NC, N, 128), jnp.float32),
  )(x, sum1, degp, Ws, Wn, b)


def _tc_layer2(hst, s2a, s2b, degp, Ws2a, Ws2b, Wn2a, Wn2b, b):
  def body(h_ref, sa_ref, sb_ref, d_ref, wsa_ref, wsb_ref, wna_ref, wnb_ref,
           b_ref, o_ref):
    deg = d_ref[0, :, :1] + d_ref[1, :, :1]
    recip = 1.0 / jnp.maximum(deg, 1.0)
    acc = jnp.dot(h_ref[0], wsa_ref[...], preferred_element_type=jnp.float32,
                   precision=lax.Precision.HIGHEST)
    acc = acc + jnp.dot(h_ref[1], wsb_ref[...],
                        preferred_element_type=jnp.float32,
                   precision=lax.Precision.HIGHEST)
    sa = sa_ref[0] + sa_ref[1]
    sb = sb_ref[0] + sb_ref[1]
    nei = jnp.dot(sa, wna_ref[...], preferred_element_type=jnp.float32,
                   precision=lax.Precision.HIGHEST)
    nei = nei + jnp.dot(sb, wnb_ref[...], preferred_element_type=jnp.float32,
                   precision=lax.Precision.HIGHEST)
    o_ref[...] = acc + nei * recip + b_ref[...]

  grid = (N // R_TC,)
  return pl.pallas_call(
      body,
      grid=grid,
      in_specs=[
          pl.BlockSpec((NC, R_TC, 128), lambda i: (0, i, 0)),
          pl.BlockSpec((NC, R_TC, 128), lambda i: (0, i, 0)),
          pl.BlockSpec((NC, R_TC, 128), lambda i: (0, i, 0)),
          pl.BlockSpec((NC, R_TC, 128), lambda i: (0, i, 0)),
          pl.BlockSpec((128, OUT_DIM), lambda i: (0, 0)),
          pl.BlockSpec((128, OUT_DIM), lambda i: (0, 0)),
          pl.BlockSpec((128, OUT_DIM), lambda i: (0, 0)),
          pl.BlockSpec((128, OUT_DIM), lambda i: (0, 0)),
          pl.BlockSpec((1, OUT_DIM), lambda i: (0, 0)),
      ],
      out_specs=pl.BlockSpec((R_TC, OUT_DIM), lambda i: (i, 0)),
      out_shape=jax.ShapeDtypeStruct((N, OUT_DIM), jnp.float32),
  )(hst, s2a, s2b, degp, Ws2a, Ws2b, Wn2a, Wn2b, b)


def kernel(feats, edge_index, W_self1, W_neigh1, b1, W_self2, W_neigh2, b2):
  src = edge_index[0]
  dst = edge_index[1]
  pad = E_PAD - E
  src_p = jnp.concatenate([src, jnp.zeros((pad,), jnp.int32)])
  dst_p = jnp.concatenate([dst, jnp.full((pad,), TRASH, jnp.int32)])

  srcs = src_p.reshape(NW * KPT, CHUNK)
  dsts = dst_p.reshape(NW * KPT, CHUNK)
  srcs_hi = srcs + N

  degp = _sc_segsum(with_gather=False)(dsts)
  sum1 = _sc_segsum()(feats, srcs, dsts)

  hst = _tc_layer1(feats, sum1, degp, W_self1, W_neigh1, b1.reshape(1, H_DIM))

  table2 = hst.reshape(2 * N, 128)
  s2a = _sc_segsum()(table2, srcs, dsts)
  s2b = _sc_segsum()(table2, srcs_hi, dsts)

  out = _tc_layer2(hst, s2a, s2b, degp,
                   W_self2[:128], W_self2[128:],
                   W_neigh2[:128], W_neigh2[128:],
                   b2.reshape(1, OUT_DIM))
  return out
